# baseline (speedup 1.0000x reference)
import dataclasses
import functools

import jax
import jax.numpy as jnp
from jax import lax
from jax.experimental import pallas as pl
from jax.experimental.pallas import tpu as pltpu
from jax.experimental.pallas import tpu_sc as plsc

N = 10000
E = 160000
D_IN = 256
D_HID = 256
D_OUT = 128
G = 64

NC = 2
NS = 16
L = 16
W = 64
NPAD = 10240
RPS = NPAD // NS
E1 = 163840
WIN1 = E1 // (NS * W)
WIN2 = E1 // (NC * NS * W)
IC = 8
NCH1 = WIN1 // IC
NCH2 = WIN2 // IC

_f32 = jnp.float32
_MESH = plsc.VectorSubcoreMesh(core_axis_name="c", subcore_axis_name="s")

_SC_CP = pltpu.CompilerParams()
if "needs_layout_passes" in pltpu.CompilerParams.__dataclass_fields__:
  _SC_CP = dataclasses.replace(_SC_CP, needs_layout_passes=False)


def _sc_agg1(featLR, srcw, dstw, zrows, zcnt):
  @functools.partial(
      pl.kernel,
      mesh=_MESH,
      out_type=[
          jax.ShapeDtypeStruct((2 * NPAD, 128), _f32),
          jax.ShapeDtypeStruct((NS, NPAD), _f32),
      ],
      scratch_types=[
          pltpu.VMEM_SHARED((NPAD, 128), _f32),
          pltpu.VMEM((IC, W), jnp.int32),
          pltpu.VMEM((IC, W), jnp.int32),
          pltpu.VMEM((W, 128), _f32),
          pltpu.VMEM((W, 128), _f32),
          pltpu.VMEM((NPAD,), _f32),
          pltpu.SemaphoreType.DMA,
      ],
      compiler_params=_SC_CP,
  )
  def k(featLR_hbm, srcw_hbm, dstw_hbm, z_hbm, zc_hbm,
        sums_hbm, cnt_hbm,
        acc, src_v, dst_v, buf0, buf1, cbuf, sem):
    c = lax.axis_index("c")
    s = lax.axis_index("s")
    sl = pl.ds(s * RPS, RPS)
    pltpu.sync_copy(z_hbm, acc.at[sl])

    @pl.when(c == 0)
    def _():
      pltpu.sync_copy(zc_hbm, cbuf)

    plsc.subcore_barrier()
    one16 = jnp.ones((L,), _f32)

    def fire(j, buf):
      pltpu.async_copy(featLR_hbm.at[src_v.at[j]], buf, sem)

    def wait(buf):
      pltpu.make_async_copy(featLR_hbm.at[src_v.at[0]], buf, sem).wait()

    def work(j, buf):
      pltpu.sync_copy(buf, acc.at[dst_v.at[j]], add=True)

      @pl.when(c == 0)
      def _():
        @pl.loop(0, W // L)
        def _(kk):
          idx = dst_v[j, pl.ds(kk * L, L)]
          plsc.addupdate_scatter(cbuf, [idx], one16)

    @pl.loop(0, NCH1)
    def _(ci):
      pltpu.sync_copy(srcw_hbm.at[c, s, ci], src_v)
      pltpu.sync_copy(dstw_hbm.at[s, ci], dst_v)
      fire(0, buf0)

      @pl.loop(0, IC, step=2)
      def _(j):
        wait(buf0)
        fire(j + 1, buf1)
        work(j, buf0)
        wait(buf1)

        @pl.when(j + 2 < IC)
        def _():
          fire(j + 2, buf0)

        work(j + 1, buf1)

    plsc.subcore_barrier()
    pltpu.sync_copy(acc.at[sl], sums_hbm.at[pl.ds(c * NPAD + s * RPS, RPS)])

    @pl.when(c == 0)
    def _():
      pltpu.sync_copy(cbuf, cnt_hbm.at[s])

  return k(featLR, srcw, dstw, zrows, zcnt)


def _sc_agg2(p2, srcw, dstw, zrows):
  @functools.partial(
      pl.kernel,
      mesh=_MESH,
      out_type=jax.ShapeDtypeStruct((2 * NPAD, 128), _f32),
      scratch_types=[
          pltpu.VMEM_SHARED((NPAD, 128), _f32),
          pltpu.VMEM((IC, W), jnp.int32),
          pltpu.VMEM((IC, W), jnp.int32),
          pltpu.VMEM((W, 128), _f32),
          pltpu.VMEM((W, 128), _f32),
          pltpu.SemaphoreType.DMA,
      ],
  )
  def k(p2_hbm, srcw_hbm, dstw_hbm, z_hbm,
        sums_hbm,
        acc, src_v, dst_v, buf0, buf1, sem):
    c = lax.axis_index("c")
    s = lax.axis_index("s")
    wid = s * NC + c
    sl = pl.ds(s * RPS, RPS)
    pltpu.sync_copy(z_hbm, acc.at[sl])
    plsc.subcore_barrier()

    def fire(j, buf):
      pltpu.async_copy(p2_hbm.at[src_v.at[j]], buf, sem)

    def wait(buf):
      pltpu.make_async_copy(p2_hbm.at[src_v.at[0]], buf, sem).wait()

    @pl.loop(0, NCH2)
    def _(ci):
      pltpu.sync_copy(srcw_hbm.at[wid, ci], src_v)
      pltpu.sync_copy(dstw_hbm.at[wid, ci], dst_v)
      fire(0, buf0)

      @pl.loop(0, IC, step=2)
      def _(j):
        wait(buf0)
        fire(j + 1, buf1)
        pltpu.sync_copy(buf0, acc.at[dst_v.at[j]], add=True)
        wait(buf1)

        @pl.when(j + 2 < IC)
        def _():
          fire(j + 2, buf0)

        pltpu.sync_copy(buf1, acc.at[dst_v.at[j + 1]], add=True)

    plsc.subcore_barrier()
    pltpu.sync_copy(acc.at[sl], sums_hbm.at[pl.ds(c * NPAD + s * RPS, RPS)])

  return k(p2, srcw, dstw, zrows)


_R1 = 1024


def _tc_dense1(sumL, sumR, cntT, feat, w1lT, w1rT, b1, w2lT, w2rT, b2):
  def body(sumL_ref, sumR_ref, cntT_ref, feat_ref, w1lT_ref, w1rT_ref, b1_ref,
           w2lT_ref, w2rT_ref, b2_ref, p2_ref, hr_ref):
    cnt = jnp.sum(cntT_ref[...], axis=1, keepdims=True)
    rc = 1.0 / jnp.maximum(cnt, 1.0)
    h = (jnp.dot(sumL_ref[...] * rc, w1lT_ref[0:128, :],
                 preferred_element_type=_f32)
         + jnp.dot(sumR_ref[...] * rc, w1lT_ref[128:256, :],
                   preferred_element_type=_f32)
         + jnp.dot(feat_ref[...], w1rT_ref[...], preferred_element_type=_f32)
         + b1_ref[...])
    nrm = jnp.sqrt(jnp.sum(h * h, axis=1, keepdims=True))
    h = h / jnp.maximum(nrm, 1e-12)
    h = jnp.maximum(h, 0.0)
    p2_ref[...] = jnp.dot(h, w2lT_ref[...], preferred_element_type=_f32)
    hr_ref[...] = (jnp.dot(h, w2rT_ref[...], preferred_element_type=_f32)
                   + b2_ref[...])

  grid = (NPAD // _R1,)
  row = lambda i: (i, 0)
  full = lambda i: (0, 0)
  return pl.pallas_call(
      body,
      grid=grid,
      in_specs=[
          pl.BlockSpec((_R1, 128), row),
          pl.BlockSpec((_R1, 128), row),
          pl.BlockSpec((_R1, NS), row),
          pl.BlockSpec((_R1, D_IN), row),
          pl.BlockSpec((D_IN, D_HID), full),
          pl.BlockSpec((D_IN, D_HID), full),
          pl.BlockSpec((1, D_HID), full),
          pl.BlockSpec((D_HID, D_OUT), full),
          pl.BlockSpec((D_HID, D_OUT), full),
          pl.BlockSpec((1, D_OUT), full),
      ],
      out_specs=[
          pl.BlockSpec((_R1, D_OUT), row),
          pl.BlockSpec((_R1, D_OUT), row),
      ],
      out_shape=[
          jax.ShapeDtypeStruct((NPAD, D_OUT), _f32),
          jax.ShapeDtypeStruct((NPAD, D_OUT), _f32),
      ],
  )(sumL, sumR, cntT, feat, w1lT, w1rT, b1, w2lT, w2rT, b2)


def _tc_dense2(s2a, s2b, cntT, hr, batchf, fcWT, fcb):
  steps = NPAD // _R1

  def body(s2a_ref, s2b_ref, cntT_ref, hr_ref, batch_ref, fcWT_ref, fcb_ref,
           out_ref, psum, pcnt):
    i = pl.program_id(0)

    @pl.when(i == 0)
    def _():
      psum[...] = jnp.zeros_like(psum)
      pcnt[...] = jnp.zeros_like(pcnt)

    cnt = jnp.sum(cntT_ref[...], axis=1, keepdims=True)
    rc = 1.0 / jnp.maximum(cnt, 1.0)
    h2 = (s2a_ref[...] + s2b_ref[...]) * rc + hr_ref[...]
    nrm = jnp.sqrt(jnp.sum(h2 * h2, axis=1, keepdims=True))
    h2 = h2 / jnp.maximum(nrm, 1e-12)
    gids = lax.broadcasted_iota(jnp.int32, (_R1, G), 1).astype(_f32)
    oh = (batch_ref[...] == gids).astype(_f32)
    cdims = (((0,), (0,)), ((), ()))
    psum[...] += lax.dot_general(oh, h2, cdims, preferred_element_type=_f32)
    pcnt[...] += lax.dot_general(oh, jnp.ones((_R1, D_OUT), _f32), cdims,
                                 preferred_element_type=_f32)

    @pl.when(i == steps - 1)
    def _():
      pooled = psum[...] / jnp.maximum(pcnt[...], 1.0)
      logits = (jnp.dot(pooled, fcWT_ref[...], preferred_element_type=_f32)
                + fcb_ref[...])
      m = jnp.max(logits, axis=1, keepdims=True)
      e = jnp.exp(logits - m)
      out_ref[...] = e / jnp.sum(e, axis=1, keepdims=True)

  row = lambda i: (i, 0)
  full = lambda i: (0, 0)
  return pl.pallas_call(
      body,
      grid=(steps,),
      in_specs=[
          pl.BlockSpec((_R1, 128), row),
          pl.BlockSpec((_R1, 128), row),
          pl.BlockSpec((_R1, NS), row),
          pl.BlockSpec((_R1, D_OUT), row),
          pl.BlockSpec((_R1, 1), row),
          pl.BlockSpec((D_OUT, 2), full),
          pl.BlockSpec((1, 2), full),
      ],
      out_specs=pl.BlockSpec((G, 2), full),
      out_shape=jax.ShapeDtypeStruct((G, 2), _f32),
      scratch_shapes=[
          pltpu.VMEM((G, D_OUT), _f32),
          pltpu.VMEM((G, D_OUT), _f32),
      ],
  )(s2a, s2b, cntT, hr, batchf, fcWT, fcb)


def kernel(feat, edge_index, batch, W1_l, b1_l, W1_r, W2_l, b2_l, W2_r,
           fc_W, fc_b):
  src = edge_index[0].astype(jnp.int32)
  dst = edge_index[1].astype(jnp.int32)
  pad = E1 - E
  src_p = jnp.concatenate([src, jnp.zeros((pad,), jnp.int32)])
  dst_p = jnp.concatenate([dst, jnp.full((pad,), NPAD - 1, jnp.int32)])
  srcw1 = jnp.stack([src_p, src_p + N]).reshape(NC, NS, NCH1, IC, W)
  dstw1 = dst_p.reshape(NS, NCH1, IC, W)
  srcw2 = src_p.reshape(NC * NS, NCH2, IC, W)
  dstw2 = dst_p.reshape(NC * NS, NCH2, IC, W)
  featLR = jnp.concatenate([feat[:, :128], feat[:, 128:]], axis=0)
  featP = jnp.concatenate([feat, jnp.zeros((NPAD - N, D_IN), _f32)])
  zrows = jnp.zeros((RPS, 128), _f32)
  zcnt = jnp.zeros((NPAD,), _f32)
  batchP = jnp.concatenate(
      [batch.astype(_f32), jnp.full((NPAD - N,), float(G), _f32)])

  sums1, cnt16 = _sc_agg1(featLR, srcw1, dstw1, zrows, zcnt)
  cntT = cnt16.T
  p2, hr = _tc_dense1(sums1[:NPAD], sums1[NPAD:], cntT, featP,
                      W1_l.T, W1_r.T, b1_l.reshape(1, -1),
                      W2_l.T, W2_r.T, b2_l.reshape(1, -1))
  sums2 = _sc_agg2(p2, srcw2, dstw2, zrows)
  out = _tc_dense2(sums2[:NPAD], sums2[NPAD:], cntT, hr,
                   batchP.reshape(NPAD, 1), fc_W.T, fc_b.reshape(1, -1))
  return out


# speedup vs baseline: 1.1345x; 1.1345x over previous
import dataclasses
import functools

import jax
import jax.numpy as jnp
from jax import lax
from jax.experimental import pallas as pl
from jax.experimental.pallas import tpu as pltpu
from jax.experimental.pallas import tpu_sc as plsc

N = 10000
E = 160000
D_IN = 256
D_HID = 256
D_OUT = 128
G = 64

NC = 2
NS = 16
L = 16
W = 128
NPAD = 10240
RPS = NPAD // NS
E1 = 163840
WIN1 = E1 // (NS * W)
WIN2 = E1 // (NC * NS * W)
IC = 4
NCH1 = WIN1 // IC
NCH2 = WIN2 // IC

_f32 = jnp.float32
_MESH = plsc.VectorSubcoreMesh(core_axis_name="c", subcore_axis_name="s")

_SC_CP = pltpu.CompilerParams()
if "needs_layout_passes" in pltpu.CompilerParams.__dataclass_fields__:
  _SC_CP = dataclasses.replace(_SC_CP, needs_layout_passes=False)


def _sc_agg1(featLR, srcw, dstw, zrows):
  @functools.partial(
      pl.kernel,
      mesh=_MESH,
      out_type=jax.ShapeDtypeStruct((2 * NPAD, 128), _f32),
      scratch_types=[
          pltpu.VMEM_SHARED((NPAD, 128), _f32),
          pltpu.VMEM((IC, W), jnp.int32),
          pltpu.VMEM((IC, W), jnp.int32),
          pltpu.VMEM((W, 128), _f32),
          pltpu.VMEM((W, 128), _f32),
          pltpu.SemaphoreType.DMA,
      ],
  )
  def k(featLR_hbm, srcw_hbm, dstw_hbm, z_hbm,
        sums_hbm,
        acc, src_v, dst_v, buf0, buf1, sem):
    c = lax.axis_index("c")
    s = lax.axis_index("s")
    sl = pl.ds(s * RPS, RPS)
    pltpu.sync_copy(z_hbm, acc.at[sl])
    plsc.subcore_barrier()

    def fire(j, buf):
      pltpu.async_copy(featLR_hbm.at[src_v.at[j]], buf, sem)

    def wait(buf):
      pltpu.make_async_copy(featLR_hbm.at[src_v.at[0]], buf, sem).wait()

    @pl.loop(0, NCH1)
    def _(ci):
      pltpu.sync_copy(srcw_hbm.at[c, s, ci], src_v)
      pltpu.sync_copy(dstw_hbm.at[s, ci], dst_v)
      fire(0, buf0)

      @pl.loop(0, IC, step=2)
      def _(j):
        wait(buf0)
        fire(j + 1, buf1)
        pltpu.sync_copy(buf0, acc.at[dst_v.at[j]], add=True)
        wait(buf1)

        @pl.when(j + 2 < IC)
        def _():
          fire(j + 2, buf0)

        pltpu.sync_copy(buf1, acc.at[dst_v.at[j + 1]], add=True)

    plsc.subcore_barrier()
    pltpu.sync_copy(acc.at[sl], sums_hbm.at[pl.ds(c * NPAD + s * RPS, RPS)])

  return k(featLR, srcw, dstw, zrows)


def _sc_counts(dstw, zcnt):
  @functools.partial(
      pl.kernel,
      mesh=_MESH,
      out_type=jax.ShapeDtypeStruct((NC * NS, NPAD), _f32),
      scratch_types=[
          pltpu.VMEM((IC, W), jnp.int32),
          pltpu.VMEM((NPAD,), _f32),
      ],
      compiler_params=_SC_CP,
  )
  def k(dstw_hbm, zc_hbm, cnt_hbm, dst_v, cbuf):
    c = lax.axis_index("c")
    s = lax.axis_index("s")
    wid = s * NC + c
    pltpu.sync_copy(zc_hbm, cbuf)
    one16 = jnp.ones((L,), _f32)

    @pl.loop(0, NCH2)
    def _(ci):
      pltpu.sync_copy(dstw_hbm.at[wid, ci], dst_v)

      @pl.loop(0, IC)
      def _(j):
        @pl.loop(0, W // L)
        def _(kk):
          idx = dst_v[j, pl.ds(kk * L, L)]
          plsc.addupdate_scatter(cbuf, [idx], one16)

    pltpu.sync_copy(cbuf, cnt_hbm.at[wid])

  return k(dstw, zcnt)


def _sc_agg2(p2, srcw, dstw, zrows):
  @functools.partial(
      pl.kernel,
      mesh=_MESH,
      out_type=jax.ShapeDtypeStruct((2 * NPAD, 128), _f32),
      scratch_types=[
          pltpu.VMEM_SHARED((NPAD, 128), _f32),
          pltpu.VMEM((IC, W), jnp.int32),
          pltpu.VMEM((IC, W), jnp.int32),
          pltpu.VMEM((W, 128), _f32),
          pltpu.VMEM((W, 128), _f32),
          pltpu.SemaphoreType.DMA,
      ],
  )
  def k(p2_hbm, srcw_hbm, dstw_hbm, z_hbm,
        sums_hbm,
        acc, src_v, dst_v, buf0, buf1, sem):
    c = lax.axis_index("c")
    s = lax.axis_index("s")
    wid = s * NC + c
    sl = pl.ds(s * RPS, RPS)
    pltpu.sync_copy(z_hbm, acc.at[sl])
    plsc.subcore_barrier()

    def fire(j, buf):
      pltpu.async_copy(p2_hbm.at[src_v.at[j]], buf, sem)

    def wait(buf):
      pltpu.make_async_copy(p2_hbm.at[src_v.at[0]], buf, sem).wait()

    @pl.loop(0, NCH2)
    def _(ci):
      pltpu.sync_copy(srcw_hbm.at[wid, ci], src_v)
      pltpu.sync_copy(dstw_hbm.at[wid, ci], dst_v)
      fire(0, buf0)

      @pl.loop(0, IC, step=2)
      def _(j):
        wait(buf0)
        fire(j + 1, buf1)
        pltpu.sync_copy(buf0, acc.at[dst_v.at[j]], add=True)
        wait(buf1)

        @pl.when(j + 2 < IC)
        def _():
          fire(j + 2, buf0)

        pltpu.sync_copy(buf1, acc.at[dst_v.at[j + 1]], add=True)

    plsc.subcore_barrier()
    pltpu.sync_copy(acc.at[sl], sums_hbm.at[pl.ds(c * NPAD + s * RPS, RPS)])

  return k(p2, srcw, dstw, zrows)


_R1 = 1024


def _tc_dense1(sumL, sumR, cntT, feat, w1lT, w1rT, b1, w2lT, w2rT, b2):
  def body(sumL_ref, sumR_ref, cntT_ref, feat_ref, w1lT_ref, w1rT_ref, b1_ref,
           w2lT_ref, w2rT_ref, b2_ref, p2_ref, hr_ref):
    cnt = jnp.sum(cntT_ref[...], axis=1, keepdims=True)
    rc = 1.0 / jnp.maximum(cnt, 1.0)
    h = (jnp.dot(sumL_ref[...] * rc, w1lT_ref[0:128, :],
                 preferred_element_type=_f32)
         + jnp.dot(sumR_ref[...] * rc, w1lT_ref[128:256, :],
                   preferred_element_type=_f32)
         + jnp.dot(feat_ref[...], w1rT_ref[...], preferred_element_type=_f32)
         + b1_ref[...])
    nrm = jnp.sqrt(jnp.sum(h * h, axis=1, keepdims=True))
    h = h / jnp.maximum(nrm, 1e-12)
    h = jnp.maximum(h, 0.0)
    p2_ref[...] = jnp.dot(h, w2lT_ref[...], preferred_element_type=_f32)
    hr_ref[...] = (jnp.dot(h, w2rT_ref[...], preferred_element_type=_f32)
                   + b2_ref[...])

  grid = (NPAD // _R1,)
  row = lambda i: (i, 0)
  full = lambda i: (0, 0)
  return pl.pallas_call(
      body,
      grid=grid,
      in_specs=[
          pl.BlockSpec((_R1, 128), row),
          pl.BlockSpec((_R1, 128), row),
          pl.BlockSpec((_R1, NC * NS), row),
          pl.BlockSpec((_R1, D_IN), row),
          pl.BlockSpec((D_IN, D_HID), full),
          pl.BlockSpec((D_IN, D_HID), full),
          pl.BlockSpec((1, D_HID), full),
          pl.BlockSpec((D_HID, D_OUT), full),
          pl.BlockSpec((D_HID, D_OUT), full),
          pl.BlockSpec((1, D_OUT), full),
      ],
      out_specs=[
          pl.BlockSpec((_R1, D_OUT), row),
          pl.BlockSpec((_R1, D_OUT), row),
      ],
      out_shape=[
          jax.ShapeDtypeStruct((NPAD, D_OUT), _f32),
          jax.ShapeDtypeStruct((NPAD, D_OUT), _f32),
      ],
  )(sumL, sumR, cntT, feat, w1lT, w1rT, b1, w2lT, w2rT, b2)


def _tc_dense2(s2a, s2b, cntT, hr, batchf, fcWT, fcb):
  steps = NPAD // _R1

  def body(s2a_ref, s2b_ref, cntT_ref, hr_ref, batch_ref, fcWT_ref, fcb_ref,
           out_ref, psum, pcnt):
    i = pl.program_id(0)

    @pl.when(i == 0)
    def _():
      psum[...] = jnp.zeros_like(psum)
      pcnt[...] = jnp.zeros_like(pcnt)

    cnt = jnp.sum(cntT_ref[...], axis=1, keepdims=True)
    rc = 1.0 / jnp.maximum(cnt, 1.0)
    h2 = (s2a_ref[...] + s2b_ref[...]) * rc + hr_ref[...]
    nrm = jnp.sqrt(jnp.sum(h2 * h2, axis=1, keepdims=True))
    h2 = h2 / jnp.maximum(nrm, 1e-12)
    gids = lax.broadcasted_iota(jnp.int32, (_R1, G), 1).astype(_f32)
    oh = (batch_ref[...] == gids).astype(_f32)
    cdims = (((0,), (0,)), ((), ()))
    psum[...] += lax.dot_general(oh, h2, cdims, preferred_element_type=_f32)
    pcnt[...] += lax.dot_general(oh, jnp.ones((_R1, D_OUT), _f32), cdims,
                                 preferred_element_type=_f32)

    @pl.when(i == steps - 1)
    def _():
      pooled = psum[...] / jnp.maximum(pcnt[...], 1.0)
      logits = (jnp.dot(pooled, fcWT_ref[...], preferred_element_type=_f32)
                + fcb_ref[...])
      m = jnp.max(logits, axis=1, keepdims=True)
      e = jnp.exp(logits - m)
      out_ref[...] = e / jnp.sum(e, axis=1, keepdims=True)

  row = lambda i: (i, 0)
  full = lambda i: (0, 0)
  return pl.pallas_call(
      body,
      grid=(steps,),
      in_specs=[
          pl.BlockSpec((_R1, 128), row),
          pl.BlockSpec((_R1, 128), row),
          pl.BlockSpec((_R1, NC * NS), row),
          pl.BlockSpec((_R1, D_OUT), row),
          pl.BlockSpec((_R1, 1), row),
          pl.BlockSpec((D_OUT, 2), full),
          pl.BlockSpec((1, 2), full),
      ],
      out_specs=pl.BlockSpec((G, 2), full),
      out_shape=jax.ShapeDtypeStruct((G, 2), _f32),
      scratch_shapes=[
          pltpu.VMEM((G, D_OUT), _f32),
          pltpu.VMEM((G, D_OUT), _f32),
      ],
  )(s2a, s2b, cntT, hr, batchf, fcWT, fcb)


def kernel(feat, edge_index, batch, W1_l, b1_l, W1_r, W2_l, b2_l, W2_r,
           fc_W, fc_b):
  src = edge_index[0].astype(jnp.int32)
  dst = edge_index[1].astype(jnp.int32)
  pad = E1 - E
  src_p = jnp.concatenate([src, jnp.zeros((pad,), jnp.int32)])
  dst_p = jnp.concatenate([dst, jnp.full((pad,), NPAD - 1, jnp.int32)])
  srcw1 = jnp.stack([2 * src_p, 2 * src_p + 1]).reshape(NC, NS, NCH1, IC, W)
  dstw1 = dst_p.reshape(NS, NCH1, IC, W)
  srcw2 = src_p.reshape(NC * NS, NCH2, IC, W)
  dstw2 = dst_p.reshape(NC * NS, NCH2, IC, W)
  featLR = feat.reshape(2 * N, 128)
  featP = jnp.concatenate([feat, jnp.zeros((NPAD - N, D_IN), _f32)])
  zrows = jnp.zeros((RPS, 128), _f32)
  zcnt = jnp.zeros((NPAD,), _f32)
  batchP = jnp.concatenate(
      [batch.astype(_f32), jnp.full((NPAD - N,), float(G), _f32)])

  sums1 = _sc_agg1(featLR, srcw1, dstw1, zrows)
  cnt32 = _sc_counts(dstw2, zcnt)
  cntT = cnt32.T
  p2, hr = _tc_dense1(sums1[:NPAD], sums1[NPAD:], cntT, featP,
                      W1_l.T, W1_r.T, b1_l.reshape(1, -1),
                      W2_l.T, W2_r.T, b2_l.reshape(1, -1))
  sums2 = _sc_agg2(p2, srcw2, dstw2, zrows)
  out = _tc_dense2(sums2[:NPAD], sums2[NPAD:], cntT, hr,
                   batchP.reshape(NPAD, 1), fc_W.T, fc_b.reshape(1, -1))
  return out


# speedup vs baseline: 1.1928x; 1.0514x over previous
import dataclasses
import functools

import jax
import jax.numpy as jnp
from jax import lax
from jax.experimental import pallas as pl
from jax.experimental.pallas import tpu as pltpu
from jax.experimental.pallas import tpu_sc as plsc

N = 10000
E = 160000
D_IN = 256
D_HID = 256
D_OUT = 128
G = 64

NC = 2
NS = 16
L = 16
W = 128
NPAD = 10240
RPS = NPAD // NS
E1 = 163840
WIN1 = E1 // (NS * W)
WIN2 = E1 // (NC * NS * W)
IC = 4
NCH1 = WIN1 // IC
NCH2 = WIN2 // IC

_f32 = jnp.float32
_bf16 = jnp.bfloat16
_MESH = plsc.VectorSubcoreMesh(core_axis_name="c", subcore_axis_name="s")

_SC_CP = pltpu.CompilerParams()
if "needs_layout_passes" in pltpu.CompilerParams.__dataclass_fields__:
  _SC_CP = dataclasses.replace(_SC_CP, needs_layout_passes=False)


def _sc_agg1(featLR, srcw, dstw, zrows):
  @functools.partial(
      pl.kernel,
      mesh=_MESH,
      out_type=jax.ShapeDtypeStruct((2 * NPAD, 128), _f32),
      scratch_types=[
          pltpu.VMEM_SHARED((NPAD, 128), _f32),
          pltpu.VMEM((IC, W), jnp.int32),
          pltpu.VMEM((IC, W), jnp.int32),
          pltpu.VMEM((W, 128), _f32),
          pltpu.VMEM((W, 128), _f32),
          pltpu.SemaphoreType.DMA,
      ],
  )
  def k(featLR_hbm, srcw_hbm, dstw_hbm, z_hbm,
        sums_hbm,
        acc, src_v, dst_v, buf0, buf1, sem):
    c = lax.axis_index("c")
    s = lax.axis_index("s")
    sl = pl.ds(s * RPS, RPS)
    pltpu.sync_copy(z_hbm, acc.at[sl])
    plsc.subcore_barrier()

    def fire(j, buf):
      pltpu.async_copy(featLR_hbm.at[src_v.at[j]], buf, sem)

    def wait(buf):
      pltpu.make_async_copy(featLR_hbm.at[src_v.at[0]], buf, sem).wait()

    @pl.loop(0, NCH1)
    def _(ci):
      pltpu.sync_copy(srcw_hbm.at[c, s, ci], src_v)
      pltpu.sync_copy(dstw_hbm.at[s, ci], dst_v)
      fire(0, buf0)

      @pl.loop(0, IC, step=2)
      def _(j):
        wait(buf0)
        fire(j + 1, buf1)
        pltpu.sync_copy(buf0, acc.at[dst_v.at[j]], add=True)
        wait(buf1)

        @pl.when(j + 2 < IC)
        def _():
          fire(j + 2, buf0)

        pltpu.sync_copy(buf1, acc.at[dst_v.at[j + 1]], add=True)

    plsc.subcore_barrier()
    pltpu.sync_copy(acc.at[sl], sums_hbm.at[pl.ds(c * NPAD + s * RPS, RPS)])

  return k(featLR, srcw, dstw, zrows)


def _sc_counts(dstw, zcnt):
  @functools.partial(
      pl.kernel,
      mesh=_MESH,
      out_type=jax.ShapeDtypeStruct((NC * NS, NPAD), _f32),
      scratch_types=[
          pltpu.VMEM((IC, W), jnp.int32),
          pltpu.VMEM((NPAD,), _f32),
      ],
      compiler_params=_SC_CP,
  )
  def k(dstw_hbm, zc_hbm, cnt_hbm, dst_v, cbuf):
    c = lax.axis_index("c")
    s = lax.axis_index("s")
    wid = s * NC + c
    pltpu.sync_copy(zc_hbm, cbuf)
    one16 = jnp.ones((L,), _f32)

    @pl.loop(0, NCH2)
    def _(ci):
      pltpu.sync_copy(dstw_hbm.at[wid, ci], dst_v)

      @pl.loop(0, IC)
      def _(j):
        @pl.loop(0, W // L)
        def _(kk):
          idx = dst_v[j, pl.ds(kk * L, L)]
          plsc.addupdate_scatter(cbuf, [idx], one16)

    pltpu.sync_copy(cbuf, cnt_hbm.at[wid])

  return k(dstw, zcnt)


def _sc_agg2(p2, srcw, dstw, zrows):
  @functools.partial(
      pl.kernel,
      mesh=_MESH,
      out_type=jax.ShapeDtypeStruct((2 * NPAD, 128), _f32),
      scratch_types=[
          pltpu.VMEM_SHARED((NPAD, 128), _f32),
          pltpu.VMEM((IC, W), jnp.int32),
          pltpu.VMEM((IC, W), jnp.int32),
          pltpu.VMEM((W, 128), _f32),
          pltpu.VMEM((W, 128), _f32),
          pltpu.SemaphoreType.DMA,
      ],
  )
  def k(p2_hbm, srcw_hbm, dstw_hbm, z_hbm,
        sums_hbm,
        acc, src_v, dst_v, buf0, buf1, sem):
    c = lax.axis_index("c")
    s = lax.axis_index("s")
    wid = s * NC + c
    sl = pl.ds(s * RPS, RPS)
    pltpu.sync_copy(z_hbm, acc.at[sl])
    plsc.subcore_barrier()

    def fire(j, buf):
      pltpu.async_copy(p2_hbm.at[src_v.at[j]], buf, sem)

    def wait(buf):
      pltpu.make_async_copy(p2_hbm.at[src_v.at[0]], buf, sem).wait()

    @pl.loop(0, NCH2)
    def _(ci):
      pltpu.sync_copy(srcw_hbm.at[wid, ci], src_v)
      pltpu.sync_copy(dstw_hbm.at[wid, ci], dst_v)
      fire(0, buf0)

      @pl.loop(0, IC, step=2)
      def _(j):
        wait(buf0)
        fire(j + 1, buf1)
        pltpu.sync_copy(buf0, acc.at[dst_v.at[j]], add=True)
        wait(buf1)

        @pl.when(j + 2 < IC)
        def _():
          fire(j + 2, buf0)

        pltpu.sync_copy(buf1, acc.at[dst_v.at[j + 1]], add=True)

    plsc.subcore_barrier()
    pltpu.sync_copy(acc.at[sl], sums_hbm.at[pl.ds(c * NPAD + s * RPS, RPS)])

  return k(p2, srcw, dstw, zrows)


_R1 = 1024


def _tc_dense1(sums1, cntT, feat, w1lT, w1rT, b1, w2lT, w2rT, b2):
  def body(sumL_ref, sumR_ref, cntT_ref, feat_ref, w1lT_ref, w1rT_ref, b1_ref,
           w2lT_ref, w2rT_ref, b2_ref, p2_ref, hr_ref):
    cnt = jnp.sum(cntT_ref[...], axis=1, keepdims=True)
    rc = 1.0 / jnp.maximum(cnt, 1.0)
    h = (jnp.dot(sumL_ref[...] * rc, w1lT_ref[0:128, :],
                 preferred_element_type=_f32)
         + jnp.dot(sumR_ref[...] * rc, w1lT_ref[128:256, :],
                   preferred_element_type=_f32)
         + jnp.dot(feat_ref[...], w1rT_ref[...], preferred_element_type=_f32)
         + b1_ref[...])
    nrm = jnp.sqrt(jnp.sum(h * h, axis=1, keepdims=True))
    h = h / jnp.maximum(nrm, 1e-12)
    h = jnp.maximum(h, 0.0)
    p2_ref[...] = jnp.dot(h, w2lT_ref[...], preferred_element_type=_f32)
    hr_ref[...] = (jnp.dot(h, w2rT_ref[...], preferred_element_type=_f32)
                   + b2_ref[...])

  grid = (NPAD // _R1,)
  row = lambda i: (i, 0)
  full = lambda i: (0, 0)
  return pl.pallas_call(
      body,
      grid=grid,
      in_specs=[
          pl.BlockSpec((_R1, 128), row),
          pl.BlockSpec((_R1, 128), lambda i: (i + NPAD // _R1, 0)),
          pl.BlockSpec((_R1, NC * NS), row),
          pl.BlockSpec((_R1, D_IN), row),
          pl.BlockSpec((D_IN, D_HID), full),
          pl.BlockSpec((D_IN, D_HID), full),
          pl.BlockSpec((1, D_HID), full),
          pl.BlockSpec((D_HID, D_OUT), full),
          pl.BlockSpec((D_HID, D_OUT), full),
          pl.BlockSpec((1, D_OUT), full),
      ],
      out_specs=[
          pl.BlockSpec((_R1, D_OUT), row),
          pl.BlockSpec((_R1, D_OUT), row),
      ],
      out_shape=[
          jax.ShapeDtypeStruct((NPAD, D_OUT), _f32),
          jax.ShapeDtypeStruct((NPAD, D_OUT), _f32),
      ],
  )(sums1, sums1, cntT, feat, w1lT, w1rT, b1, w2lT, w2rT, b2)


def _tc_dense2(sums2, cntT, hr, batchf, fcWT, fcb):
  steps = NPAD // _R1

  def body(s2a_ref, s2b_ref, cntT_ref, hr_ref, batch_ref, fcWT_ref, fcb_ref,
           out_ref, psum, pcnt):
    i = pl.program_id(0)

    @pl.when(i == 0)
    def _():
      psum[...] = jnp.zeros_like(psum)
      pcnt[...] = jnp.zeros_like(pcnt)

    cnt = jnp.sum(cntT_ref[...], axis=1, keepdims=True)
    rc = 1.0 / jnp.maximum(cnt, 1.0)
    h2 = (s2a_ref[...] + s2b_ref[...]) * rc + hr_ref[...]
    nrm = jnp.sqrt(jnp.sum(h2 * h2, axis=1, keepdims=True))
    h2 = h2 / jnp.maximum(nrm, 1e-12)
    gids = lax.broadcasted_iota(jnp.int32, (_R1, G), 1).astype(_f32)
    oh = (batch_ref[...] == gids).astype(_f32)
    cdims = (((0,), (0,)), ((), ()))
    psum[...] += lax.dot_general(oh, h2, cdims, preferred_element_type=_f32)
    pcnt[...] += lax.dot_general(oh, jnp.ones((_R1, D_OUT), _f32), cdims,
                                 preferred_element_type=_f32)

    @pl.when(i == steps - 1)
    def _():
      pooled = psum[...] / jnp.maximum(pcnt[...], 1.0)
      logits = (jnp.dot(pooled, fcWT_ref[...], preferred_element_type=_f32)
                + fcb_ref[...])
      m = jnp.max(logits, axis=1, keepdims=True)
      e = jnp.exp(logits - m)
      out_ref[...] = e / jnp.sum(e, axis=1, keepdims=True)

  row = lambda i: (i, 0)
  full = lambda i: (0, 0)
  return pl.pallas_call(
      body,
      grid=(steps,),
      in_specs=[
          pl.BlockSpec((_R1, 128), row),
          pl.BlockSpec((_R1, 128), lambda i: (i + NPAD // _R1, 0)),
          pl.BlockSpec((_R1, NC * NS), row),
          pl.BlockSpec((_R1, D_OUT), row),
          pl.BlockSpec((_R1, 1), row),
          pl.BlockSpec((D_OUT, 2), full),
          pl.BlockSpec((1, 2), full),
      ],
      out_specs=pl.BlockSpec((G, 2), full),
      out_shape=jax.ShapeDtypeStruct((G, 2), _f32),
      scratch_shapes=[
          pltpu.VMEM((G, D_OUT), _f32),
          pltpu.VMEM((G, D_OUT), _f32),
      ],
  )(sums2, sums2, cntT, hr, batchf, fcWT, fcb)


def kernel(feat, edge_index, batch, W1_l, b1_l, W1_r, W2_l, b2_l, W2_r,
           fc_W, fc_b):
  src = edge_index[0].astype(jnp.int32)
  dst = edge_index[1].astype(jnp.int32)
  pad = E1 - E
  src_p = jnp.concatenate([src, jnp.zeros((pad,), jnp.int32)])
  dst_p = jnp.concatenate([dst, jnp.full((pad,), NPAD - 1, jnp.int32)])
  srcw1 = jnp.stack([2 * src_p, 2 * src_p + 1]).reshape(NC, NS, NCH1, IC, W)
  dstw1 = dst_p.reshape(NS, NCH1, IC, W)
  srcw2 = src_p.reshape(NC * NS, NCH2, IC, W)
  dstw2 = dst_p.reshape(NC * NS, NCH2, IC, W)
  featLR = feat.reshape(2 * N, 128)
  featP = jnp.concatenate([feat, jnp.zeros((NPAD - N, D_IN), _f32)])
  zrows = jnp.zeros((RPS, 128), _f32)
  zcnt = jnp.zeros((NPAD,), _f32)
  batchP = jnp.concatenate(
      [batch.astype(_f32), jnp.full((NPAD - N,), float(G), _f32)])

  sums1 = _sc_agg1(featLR, srcw1, dstw1, zrows)
  cnt32 = _sc_counts(dstw2, zcnt)
  cntT = cnt32.T
  p2, hr = _tc_dense1(sums1, cntT, featP,
                      W1_l.T, W1_r.T, b1_l.reshape(1, -1),
                      W2_l.T, W2_r.T, b2_l.reshape(1, -1))
  sums2 = _sc_agg2(p2, srcw2, dstw2, zrows)
  out = _tc_dense2(sums2, cntT, hr,
                   batchP.reshape(NPAD, 1), fc_W.T, fc_b.reshape(1, -1))
  return out


# speedup vs baseline: 1.2286x; 1.0300x over previous
import dataclasses
import functools

import jax
import jax.numpy as jnp
from jax import lax
from jax.experimental import pallas as pl
from jax.experimental.pallas import tpu as pltpu
from jax.experimental.pallas import tpu_sc as plsc

N = 10000
E = 160000
D_IN = 256
D_HID = 256
D_OUT = 128
G = 64

NC = 2
NS = 16
L = 16
W = 128
NPAD = 10240
RPS = NPAD // NS
E1 = 163840
WIN1 = E1 // (NS * W)
WIN2 = E1 // (NC * NS * W)
IC = 8
NCH1 = WIN1 // IC
NCH2 = WIN2 // IC

_f32 = jnp.float32
_bf16 = jnp.bfloat16
_MESH = plsc.VectorSubcoreMesh(core_axis_name="c", subcore_axis_name="s")

_SC_CP = pltpu.CompilerParams()
if "needs_layout_passes" in pltpu.CompilerParams.__dataclass_fields__:
  _SC_CP = dataclasses.replace(_SC_CP, needs_layout_passes=False)


def _sc_agg1(featLR, srcw, dstw, zrows):
  @functools.partial(
      pl.kernel,
      mesh=_MESH,
      out_type=jax.ShapeDtypeStruct((2 * NPAD, 128), _f32),
      scratch_types=[
          pltpu.VMEM_SHARED((NPAD, 128), _f32),
          pltpu.VMEM((IC, W), jnp.int32),
          pltpu.VMEM((IC, W), jnp.int32),
          pltpu.VMEM((W, 128), _f32),
          pltpu.VMEM((W, 128), _f32),
          pltpu.SemaphoreType.DMA,
      ],
  )
  def k(featLR_hbm, srcw_hbm, dstw_hbm, z_hbm,
        sums_hbm,
        acc, src_v, dst_v, buf0, buf1, sem):
    c = lax.axis_index("c")
    s = lax.axis_index("s")
    sl = pl.ds(s * RPS, RPS)
    pltpu.sync_copy(z_hbm, acc.at[sl])
    plsc.subcore_barrier()

    def fire(j, buf):
      pltpu.async_copy(featLR_hbm.at[src_v.at[j]], buf, sem)

    def wait(buf):
      pltpu.make_async_copy(featLR_hbm.at[src_v.at[0]], buf, sem).wait()

    @pl.loop(0, NCH1)
    def _(ci):
      pltpu.sync_copy(srcw_hbm.at[c, s, ci], src_v)
      pltpu.sync_copy(dstw_hbm.at[s, ci], dst_v)
      fire(0, buf0)

      @pl.loop(0, IC, step=2)
      def _(j):
        wait(buf0)
        fire(j + 1, buf1)
        pltpu.sync_copy(buf0, acc.at[dst_v.at[j]], add=True)
        wait(buf1)

        @pl.when(j + 2 < IC)
        def _():
          fire(j + 2, buf0)

        pltpu.sync_copy(buf1, acc.at[dst_v.at[j + 1]], add=True)

    plsc.subcore_barrier()
    pltpu.sync_copy(acc.at[sl], sums_hbm.at[pl.ds(c * NPAD + s * RPS, RPS)])

  return k(featLR, srcw, dstw, zrows)


def _sc_counts(dstw, zcnt):
  @functools.partial(
      pl.kernel,
      mesh=_MESH,
      out_type=jax.ShapeDtypeStruct((NC * NS, NPAD), _f32),
      scratch_types=[
          pltpu.VMEM((IC, W), jnp.int32),
          pltpu.VMEM((NPAD,), _f32),
      ],
      compiler_params=_SC_CP,
  )
  def k(dstw_hbm, zc_hbm, cnt_hbm, dst_v, cbuf):
    c = lax.axis_index("c")
    s = lax.axis_index("s")
    wid = s * NC + c
    pltpu.sync_copy(zc_hbm, cbuf)
    one16 = jnp.ones((L,), _f32)

    @pl.loop(0, NCH2)
    def _(ci):
      pltpu.sync_copy(dstw_hbm.at[wid, ci], dst_v)

      @pl.loop(0, IC)
      def _(j):
        @pl.loop(0, W // L)
        def _(kk):
          idx = dst_v[j, pl.ds(kk * L, L)]
          plsc.addupdate_scatter(cbuf, [idx], one16)

    pltpu.sync_copy(cbuf, cnt_hbm.at[wid])

  return k(dstw, zcnt)


def _sc_agg2(p2, srcw, dstw, zrows):
  @functools.partial(
      pl.kernel,
      mesh=_MESH,
      out_type=jax.ShapeDtypeStruct((2 * NPAD, 128), _f32),
      scratch_types=[
          pltpu.VMEM_SHARED((NPAD, 128), _f32),
          pltpu.VMEM((IC, W), jnp.int32),
          pltpu.VMEM((IC, W), jnp.int32),
          pltpu.VMEM((W, 128), _f32),
          pltpu.VMEM((W, 128), _f32),
          pltpu.SemaphoreType.DMA,
      ],
  )
  def k(p2_hbm, srcw_hbm, dstw_hbm, z_hbm,
        sums_hbm,
        acc, src_v, dst_v, buf0, buf1, sem):
    c = lax.axis_index("c")
    s = lax.axis_index("s")
    wid = s * NC + c
    sl = pl.ds(s * RPS, RPS)
    pltpu.sync_copy(z_hbm, acc.at[sl])
    plsc.subcore_barrier()

    def fire(j, buf):
      pltpu.async_copy(p2_hbm.at[src_v.at[j]], buf, sem)

    def wait(buf):
      pltpu.make_async_copy(p2_hbm.at[src_v.at[0]], buf, sem).wait()

    @pl.loop(0, NCH2)
    def _(ci):
      pltpu.sync_copy(srcw_hbm.at[wid, ci], src_v)
      pltpu.sync_copy(dstw_hbm.at[wid, ci], dst_v)
      fire(0, buf0)

      @pl.loop(0, IC, step=2)
      def _(j):
        wait(buf0)
        fire(j + 1, buf1)
        pltpu.sync_copy(buf0, acc.at[dst_v.at[j]], add=True)
        wait(buf1)

        @pl.when(j + 2 < IC)
        def _():
          fire(j + 2, buf0)

        pltpu.sync_copy(buf1, acc.at[dst_v.at[j + 1]], add=True)

    plsc.subcore_barrier()
    pltpu.sync_copy(acc.at[sl], sums_hbm.at[pl.ds(c * NPAD + s * RPS, RPS)])

  return k(p2, srcw, dstw, zrows)


_R1 = 1024


def _tc_dense1(sums1, cntT, feat, w1lT, w1rT, b1, w2lT, w2rT, b2):
  def body(sumL_ref, sumR_ref, cntT_ref, feat_ref, w1lT_ref, w1rT_ref, b1_ref,
           w2lT_ref, w2rT_ref, b2_ref, p2_ref, hr_ref):
    cnt = jnp.sum(cntT_ref[...], axis=1, keepdims=True)
    rc = 1.0 / jnp.maximum(cnt, 1.0)
    h = (jnp.dot(sumL_ref[...] * rc, w1lT_ref[0:128, :],
                 preferred_element_type=_f32)
         + jnp.dot(sumR_ref[...] * rc, w1lT_ref[128:256, :],
                   preferred_element_type=_f32)
         + jnp.dot(feat_ref[...], w1rT_ref[...], preferred_element_type=_f32)
         + b1_ref[...])
    nrm = jnp.sqrt(jnp.sum(h * h, axis=1, keepdims=True))
    h = h / jnp.maximum(nrm, 1e-12)
    h = jnp.maximum(h, 0.0)
    p2_ref[...] = jnp.dot(h, w2lT_ref[...], preferred_element_type=_f32)
    hr_ref[...] = (jnp.dot(h, w2rT_ref[...], preferred_element_type=_f32)
                   + b2_ref[...])

  grid = (NPAD // _R1,)
  row = lambda i: (i, 0)
  full = lambda i: (0, 0)
  return pl.pallas_call(
      body,
      grid=grid,
      in_specs=[
          pl.BlockSpec((_R1, 128), row),
          pl.BlockSpec((_R1, 128), lambda i: (i + NPAD // _R1, 0)),
          pl.BlockSpec((_R1, NC * NS), row),
          pl.BlockSpec((_R1, D_IN), row),
          pl.BlockSpec((D_IN, D_HID), full),
          pl.BlockSpec((D_IN, D_HID), full),
          pl.BlockSpec((1, D_HID), full),
          pl.BlockSpec((D_HID, D_OUT), full),
          pl.BlockSpec((D_HID, D_OUT), full),
          pl.BlockSpec((1, D_OUT), full),
      ],
      out_specs=[
          pl.BlockSpec((_R1, D_OUT), row),
          pl.BlockSpec((_R1, D_OUT), row),
      ],
      out_shape=[
          jax.ShapeDtypeStruct((NPAD, D_OUT), _f32),
          jax.ShapeDtypeStruct((NPAD, D_OUT), _f32),
      ],
  )(sums1, sums1, cntT, feat, w1lT, w1rT, b1, w2lT, w2rT, b2)


def _tc_dense2(sums2, cntT, hr, batchf, fcWT, fcb):
  steps = NPAD // _R1

  def body(s2a_ref, s2b_ref, cntT_ref, hr_ref, batch_ref, fcWT_ref, fcb_ref,
           out_ref, psum, pcnt):
    i = pl.program_id(0)

    @pl.when(i == 0)
    def _():
      psum[...] = jnp.zeros_like(psum)
      pcnt[...] = jnp.zeros_like(pcnt)

    cnt = jnp.sum(cntT_ref[...], axis=1, keepdims=True)
    rc = 1.0 / jnp.maximum(cnt, 1.0)
    h2 = (s2a_ref[...] + s2b_ref[...]) * rc + hr_ref[...]
    nrm = jnp.sqrt(jnp.sum(h2 * h2, axis=1, keepdims=True))
    h2 = h2 / jnp.maximum(nrm, 1e-12)
    gids = lax.broadcasted_iota(jnp.int32, (_R1, G), 1).astype(_f32)
    oh = (batch_ref[...] == gids).astype(_f32)
    cdims = (((0,), (0,)), ((), ()))
    psum[...] += lax.dot_general(oh, h2, cdims, preferred_element_type=_f32)
    pcnt[...] += lax.dot_general(oh, jnp.ones((_R1, D_OUT), _f32), cdims,
                                 preferred_element_type=_f32)

    @pl.when(i == steps - 1)
    def _():
      pooled = psum[...] / jnp.maximum(pcnt[...], 1.0)
      logits = (jnp.dot(pooled, fcWT_ref[...], preferred_element_type=_f32)
                + fcb_ref[...])
      m = jnp.max(logits, axis=1, keepdims=True)
      e = jnp.exp(logits - m)
      out_ref[...] = e / jnp.sum(e, axis=1, keepdims=True)

  row = lambda i: (i, 0)
  full = lambda i: (0, 0)
  return pl.pallas_call(
      body,
      grid=(steps,),
      in_specs=[
          pl.BlockSpec((_R1, 128), row),
          pl.BlockSpec((_R1, 128), lambda i: (i + NPAD // _R1, 0)),
          pl.BlockSpec((_R1, NC * NS), row),
          pl.BlockSpec((_R1, D_OUT), row),
          pl.BlockSpec((_R1, 1), row),
          pl.BlockSpec((D_OUT, 2), full),
          pl.BlockSpec((1, 2), full),
      ],
      out_specs=pl.BlockSpec((G, 2), full),
      out_shape=jax.ShapeDtypeStruct((G, 2), _f32),
      scratch_shapes=[
          pltpu.VMEM((G, D_OUT), _f32),
          pltpu.VMEM((G, D_OUT), _f32),
      ],
  )(sums2, sums2, cntT, hr, batchf, fcWT, fcb)


def kernel(feat, edge_index, batch, W1_l, b1_l, W1_r, W2_l, b2_l, W2_r,
           fc_W, fc_b):
  src = edge_index[0].astype(jnp.int32)
  dst = edge_index[1].astype(jnp.int32)
  pad = E1 - E
  src_p = jnp.concatenate([src, jnp.zeros((pad,), jnp.int32)])
  dst_p = jnp.concatenate([dst, jnp.full((pad,), NPAD - 1, jnp.int32)])
  srcw1 = jnp.stack([2 * src_p, 2 * src_p + 1]).reshape(NC, NS, NCH1, IC, W)
  dstw1 = dst_p.reshape(NS, NCH1, IC, W)
  srcw2 = src_p.reshape(NC * NS, NCH2, IC, W)
  dstw2 = dst_p.reshape(NC * NS, NCH2, IC, W)
  featLR = feat.reshape(2 * N, 128)
  featP = jnp.concatenate([feat, jnp.zeros((NPAD - N, D_IN), _f32)])
  zrows = jnp.zeros((RPS, 128), _f32)
  zcnt = jnp.zeros((NPAD,), _f32)
  batchP = jnp.concatenate(
      [batch.astype(_f32), jnp.full((NPAD - N,), float(G), _f32)])

  sums1 = _sc_agg1(featLR, srcw1, dstw1, zrows)
  cnt32 = _sc_counts(dstw2, zcnt)
  cntT = cnt32.T
  p2, hr = _tc_dense1(sums1, cntT, featP,
                      W1_l.T, W1_r.T, b1_l.reshape(1, -1),
                      W2_l.T, W2_r.T, b2_l.reshape(1, -1))
  sums2 = _sc_agg2(p2, srcw2, dstw2, zrows)
  out = _tc_dense2(sums2, cntT, hr,
                   batchP.reshape(NPAD, 1), fc_W.T, fc_b.reshape(1, -1))
  return out
